# baseline (speedup 1.0000x reference)
import functools

import jax
import jax.numpy as jnp
from jax import lax
from jax.experimental import pallas as pl
from jax.experimental.pallas import tpu as pltpu
from jax.experimental.pallas import tpu_sc as plsc

_N, _K, _D, _G = 10000, 32, 128, 8
_B = 200
_NB = _N // _B
_BK = _B * _K
_EPS = 1e-5

_NC, _NS = 2, 16
_NW = _NC * _NS
_RPW = _N * _K // _NW
_CH = 80
_NIT = _RPW // _CH


def _bn_scale(s, ss, g):
    mean = s / _N
    var = ss / _N - mean * mean
    inv = g * lax.rsqrt(var + _EPS)
    return mean, inv


def _proj_qkv_body(feat_ref, wp_ref, cpad_ref, gp_ref, bp_ref, wv_ref, bv_ref,
                   wkwp_ref, bkwp_ref, wp1p_ref, v_ref, aux_ref, ccp_ref,
                   x_vm, s_vm, ss_vm):
    p = pl.program_id(0)
    i = pl.program_id(1)
    rows = pl.ds(i * _B, _B)

    @pl.when(p == 0)
    def _phase0():
        x = jnp.dot(feat_ref[...], wp_ref[...],
                    preferred_element_type=jnp.float32)
        x_vm[rows, :] = x

        @pl.when(i == 0)
        def _():
            s_vm[...] = jnp.zeros_like(s_vm)
            ss_vm[...] = jnp.zeros_like(ss_vm)

        s_vm[...] += jnp.sum(x, axis=0, keepdims=True)
        ss_vm[...] += jnp.sum(x * x, axis=0, keepdims=True)

    @pl.when(p == 1)
    def _phase1():
        mean, inv = _bn_scale(s_vm[...], ss_vm[...], gp_ref[...])
        f = jnp.maximum((x_vm[rows, :] - mean) * inv + bp_ref[...], 0.0)
        v_ref[...] = jnp.dot(f, wv_ref[...],
                             preferred_element_type=jnp.float32) + bv_ref[...]
        aux_ref[...] = (jnp.dot(f, wkwp_ref[...],
                                preferred_element_type=jnp.float32)
                        + bkwp_ref[...] + cpad_ref[...])
        ccp_ref[...] = jnp.dot(cpad_ref[...], wp1p_ref[...],
                               preferred_element_type=jnp.float32)


_NBUF = 5
_NGRP = _NIT // _NBUF


def _sc_gather(v, aux, idx):
    mesh = plsc.VectorSubcoreMesh(core_axis_name="c", subcore_axis_name="s")

    @functools.partial(
        pl.kernel,
        out_type=(jax.ShapeDtypeStruct((_N * _K, _D), jnp.float32),
                  jax.ShapeDtypeStruct((_N * _K, 16), jnp.float32)),
        mesh=mesh,
        scratch_types=[
            pltpu.VMEM((_RPW,), jnp.int32),
            pltpu.VMEM((_NBUF, _CH, _D), jnp.float32),
            pltpu.VMEM((_NBUF, _CH, 16), jnp.float32),
            pltpu.SemaphoreType.DMA((_NBUF,)),
            pltpu.SemaphoreType.DMA((_NBUF,)),
        ],
        compiler_params=pltpu.CompilerParams(use_tc_tiling_on_sc=False),
    )
    def gather_kernel(v_hbm, aux_hbm, idx_hbm, vout, aout,
                      idx_all, vbufs, abufs, sem_g, sem_s):
        wid = lax.axis_index("s") * _NC + lax.axis_index("c")
        pltpu.sync_copy(idx_hbm.at[pl.ds(wid * _RPW, _RPW)], idx_all)

        def fire(chunk, b):
            isl = idx_all.at[pl.ds(chunk * _CH, _CH)]
            pltpu.async_copy(v_hbm.at[isl], vbufs.at[b], sem_g.at[b])
            pltpu.async_copy(aux_hbm.at[isl], abufs.at[b], sem_g.at[b])

        def drain_gather(b):
            pltpu.make_async_copy(v_hbm.at[pl.ds(0, _CH)], vbufs.at[b],
                                  sem_g.at[b]).wait()
            pltpu.make_async_copy(aux_hbm.at[pl.ds(0, _CH)], abufs.at[b],
                                  sem_g.at[b]).wait()

        def scatter(chunk, b):
            base = wid * _RPW + chunk * _CH
            pltpu.async_copy(vbufs.at[b], vout.at[pl.ds(base, _CH)],
                             sem_s.at[b])
            pltpu.async_copy(abufs.at[b], aout.at[pl.ds(base, _CH)],
                             sem_s.at[b])

        def drain_scatter(b):
            pltpu.make_async_copy(vbufs.at[b], vout.at[pl.ds(0, _CH)],
                                  sem_s.at[b]).wait()
            pltpu.make_async_copy(abufs.at[b], aout.at[pl.ds(0, _CH)],
                                  sem_s.at[b]).wait()

        for b in range(_NBUF):
            fire(b, b)

        def group(g, carry):
            for b in range(_NBUF):
                drain_gather(b)
                scatter(g * _NBUF + b, b)

            @pl.when(g < _NGRP - 1)
            def _():
                for b in range(_NBUF):
                    drain_scatter(b)
                    fire((g + 1) * _NBUF + b, b)

            return carry

        lax.fori_loop(0, _NGRP, group, 0)
        for b in range(_NBUF):
            drain_scatter(b)

    return gather_kernel(v, aux, idx)


def _attn_post_body(vn_ref, auxn_ref, ccp_ref, feat_ref, wp1p_ref, bp1_ref,
                    wp2_ref, bp2_ref, ww2_ref, e8_ref, g1_ref, b1_ref, wl_ref,
                    g2_ref, b2_ref, out_ref,
                    att_vm, y_vm, as_vm, ass_vm, ys_vm, yss_vm):
    p = pl.program_id(0)
    i = pl.program_id(1)
    rows = pl.ds(i * _B, _B)

    @pl.when(p == 0)
    def _attention():
        aux = auxn_ref[...]
        ccp = ccp_ref[...]
        ccp_rep = jnp.broadcast_to(
            ccp[:, None, :], (_B, _K, _D)).reshape(_BK, _D)
        h = jnp.maximum(
            jnp.dot(aux, wp1p_ref[...], preferred_element_type=jnp.float32)
            - ccp_rep + bp1_ref[...], 0.0)
        logits = (jnp.dot(h, ww2_ref[...],
                          preferred_element_type=jnp.float32)
                  - aux[:, :_G]).reshape(_B, _K, _G)
        m = jnp.max(logits, axis=1, keepdims=True)
        e = jnp.exp(logits - m)
        denom = jnp.sum(e, axis=1)
        pe = jnp.dot(h, wp2_ref[...],
                     preferred_element_type=jnp.float32) + bp2_ref[...]
        wf = jnp.dot(e.reshape(_BK, _G), e8_ref[...],
                     preferred_element_type=jnp.float32)
        vg = vn_ref[...] + pe
        att_un = jnp.sum((wf * vg).reshape(_B, _K, _D), axis=1)
        att = att_un / jnp.dot(denom, e8_ref[...],
                               preferred_element_type=jnp.float32)
        att_vm[rows, :] = att

        @pl.when(i == 0)
        def _():
            as_vm[...] = jnp.zeros_like(as_vm)
            ass_vm[...] = jnp.zeros_like(ass_vm)

        as_vm[...] += jnp.sum(att, axis=0, keepdims=True)
        ass_vm[...] += jnp.sum(att * att, axis=0, keepdims=True)

    @pl.when(p == 1)
    def _post():
        mean, inv = _bn_scale(as_vm[...], ass_vm[...], g1_ref[...])
        f1 = jnp.maximum((att_vm[rows, :] - mean) * inv + b1_ref[...], 0.0)
        y = jnp.dot(f1, wl_ref[...], preferred_element_type=jnp.float32)
        y_vm[rows, :] = y

        @pl.when(i == 0)
        def _():
            ys_vm[...] = jnp.zeros_like(ys_vm)
            yss_vm[...] = jnp.zeros_like(yss_vm)

        ys_vm[...] += jnp.sum(y, axis=0, keepdims=True)
        yss_vm[...] += jnp.sum(y * y, axis=0, keepdims=True)

    @pl.when(p == 2)
    def _final():
        mean, inv = _bn_scale(ys_vm[...], yss_vm[...], g2_ref[...])
        out_ref[...] = jnp.maximum(
            feat_ref[...] + (y_vm[rows, :] - mean) * inv + b2_ref[...], 0.0)


def _phase_spec(bs, phase):
    return pl.BlockSpec(bs, lambda p, i: (jnp.where(p == phase, i, 0), 0))


def _out_spec(bs, phase):
    return pl.BlockSpec(bs, lambda p, i: (jnp.where(p == phase, i, 0), 0))


def _const_spec(bs):
    return pl.BlockSpec(bs, lambda p, i: (0, 0))


_ARB2 = pltpu.CompilerParams(
    dimension_semantics=("arbitrary", "arbitrary"))


def kernel(coords, features, neighbor_indices, W_proj, g_proj, b_proj,
           Wq, bq, Wk, bk, Wv, bv, Wp1, bp1, Wp2, bp2, Ww, bw,
           g1, b1, W_lin, g2, b2):
    f32 = jnp.float32
    wkw = Wk @ Ww
    wkw_pad = jnp.pad(wkw, ((0, 0), (0, 16 - _G)))
    bkw_pad = jnp.pad(bk @ Ww, (0, 16 - _G)).reshape(1, 16)
    cpad = jnp.pad(coords.astype(f32), ((0, 0), (_G, 16 - _G - 3)))
    wp1_pad = jnp.zeros((16, _D), f32).at[_G:_G + 3, :].set(Wp1)
    ww2 = Wp2 @ Ww
    e8 = (jnp.arange(_D)[None, :] // (_D // _G)
          == jnp.arange(_G)[:, None]).astype(f32)
    r = lambda a: a.reshape(1, -1)

    v, aux, ccp = pl.pallas_call(
        _proj_qkv_body,
        grid=(2, _NB),
        in_specs=[_phase_spec((_B, _D), 0), _const_spec((_D, _D)),
                  _phase_spec((_B, 16), 1), _const_spec((1, _D)),
                  _const_spec((1, _D)), _const_spec((_D, _D)),
                  _const_spec((1, _D)), _const_spec((_D, 16)),
                  _const_spec((1, 16)), _const_spec((16, _D))],
        out_specs=[_out_spec((_B, _D), 1), _out_spec((_B, 16), 1),
                   _out_spec((_B, _D), 1)],
        out_shape=[jax.ShapeDtypeStruct((_N, _D), f32),
                   jax.ShapeDtypeStruct((_N, 16), f32),
                   jax.ShapeDtypeStruct((_N, _D), f32)],
        scratch_shapes=[pltpu.VMEM((_N, _D), f32),
                        pltpu.VMEM((1, _D), f32),
                        pltpu.VMEM((1, _D), f32)],
        compiler_params=_ARB2,
    )(features, W_proj, cpad, r(g_proj), r(b_proj), Wv, r(bv), wkw_pad,
      bkw_pad, wp1_pad)

    idx = neighbor_indices.astype(jnp.int32).reshape(-1)
    vn, auxn = _sc_gather(v, aux, idx)

    out = pl.pallas_call(
        _attn_post_body,
        grid=(3, _NB),
        in_specs=[_phase_spec((_BK, _D), 0), _phase_spec((_BK, 16), 0),
                  _phase_spec((_B, _D), 0), _phase_spec((_B, _D), 2),
                  _const_spec((16, _D)), _const_spec((1, _D)),
                  _const_spec((_D, _D)), _const_spec((1, _D)),
                  _const_spec((_D, _G)), _const_spec((_G, _D)),
                  _const_spec((1, _D)), _const_spec((1, _D)),
                  _const_spec((_D, _D)), _const_spec((1, _D)),
                  _const_spec((1, _D))],
        out_specs=_out_spec((_B, _D), 2),
        out_shape=jax.ShapeDtypeStruct((_N, _D), f32),
        scratch_shapes=[pltpu.VMEM((_N, _D), f32),
                        pltpu.VMEM((_N, _D), f32),
                        pltpu.VMEM((1, _D), f32),
                        pltpu.VMEM((1, _D), f32),
                        pltpu.VMEM((1, _D), f32),
                        pltpu.VMEM((1, _D), f32)],
        compiler_params=_ARB2,
    )(vn, auxn, ccp, features, wp1_pad, r(bp1), Wp2, r(bp2), ww2, e8,
      r(g1), r(b1), W_lin, r(g2), r(b2))
    return out


# speedup vs baseline: 1.1420x; 1.1420x over previous
import functools

import jax
import jax.numpy as jnp
from jax import lax
from jax.experimental import pallas as pl
from jax.experimental.pallas import tpu as pltpu
from jax.experimental.pallas import tpu_sc as plsc

_N, _K, _D, _G = 10000, 32, 128, 8
_B = 200
_NB = _N // _B
_BK = _B * _K
_EPS = 1e-5

_NC, _NS = 2, 16
_NW = _NC * _NS
_RPW = _N * _K // _NW
_CH = 80
_NIT = _RPW // _CH


def _bn_scale(s, ss, g):
    mean = s / _N
    var = ss / _N - mean * mean
    inv = g * lax.rsqrt(var + _EPS)
    return mean, inv


def _proj_qkv_body(feat_ref, wp_ref, cpad_ref, gp_ref, bp_ref, wv_ref, bv_ref,
                   wkwp_ref, bkwp_ref, wp1p_ref, v_ref, ccp_ref,
                   x_vm, s_vm, ss_vm):
    p = pl.program_id(0)
    i = pl.program_id(1)
    rows = pl.ds(i * _B, _B)

    @pl.when(p == 0)
    def _phase0():
        x = jnp.dot(feat_ref[...], wp_ref[...],
                    preferred_element_type=jnp.float32)
        x_vm[rows, :] = x

        @pl.when(i == 0)
        def _():
            s_vm[...] = jnp.zeros_like(s_vm)
            ss_vm[...] = jnp.zeros_like(ss_vm)

        s_vm[...] += jnp.sum(x, axis=0, keepdims=True)
        ss_vm[...] += jnp.sum(x * x, axis=0, keepdims=True)

    @pl.when(p == 1)
    def _phase1():
        mean, inv = _bn_scale(s_vm[...], ss_vm[...], gp_ref[...])
        f = jnp.maximum((x_vm[rows, :] - mean) * inv + bp_ref[...], 0.0)
        v = jnp.dot(f, wv_ref[...],
                    preferred_element_type=jnp.float32) + bv_ref[...]
        hi = lax.bitcast_convert_type(v[:, :_D // 2], jnp.int32)
        lo = lax.bitcast_convert_type(v[:, _D // 2:], jnp.int32)
        rnd = jnp.int32(0x8000)
        packed = lax.bitcast_convert_type(
            ((hi + rnd) & jnp.int32(-65536))
            | (((lo + rnd) >> 16) & jnp.int32(0xFFFF)), jnp.float32)
        aux = (jnp.dot(f, wkwp_ref[...],
                       preferred_element_type=jnp.float32)
               + bkwp_ref[...] + cpad_ref[...])
        zpad = jnp.zeros((_B, _D - _D // 2 - 16), jnp.float32)
        v_ref[...] = jnp.concatenate([packed, aux, zpad], axis=-1)
        ccp_ref[...] = jnp.dot(cpad_ref[...], wp1p_ref[...],
                               preferred_element_type=jnp.float32)


_NBUF = 5
_NGRP = _NIT // _NBUF


def _sc_gather(v, idx):
    mesh = plsc.VectorSubcoreMesh(core_axis_name="c", subcore_axis_name="s")

    @functools.partial(
        pl.kernel,
        out_type=jax.ShapeDtypeStruct((_N * _K, _D), jnp.float32),
        mesh=mesh,
        scratch_types=[
            pltpu.VMEM((_RPW,), jnp.int32),
            pltpu.VMEM((_NBUF, _CH, _D), jnp.float32),
            pltpu.SemaphoreType.DMA((_NBUF,)),
            pltpu.SemaphoreType.DMA((_NBUF,)),
        ],
        compiler_params=pltpu.CompilerParams(use_tc_tiling_on_sc=False),
    )
    def gather_kernel(v_hbm, idx_hbm, vout, idx_all, vbufs, sem_g, sem_s):
        wid = lax.axis_index("s") * _NC + lax.axis_index("c")
        pltpu.sync_copy(idx_hbm.at[pl.ds(wid * _RPW, _RPW)], idx_all)

        def fire(chunk, b):
            isl = idx_all.at[pl.ds(chunk * _CH, _CH)]
            pltpu.async_copy(v_hbm.at[isl], vbufs.at[b], sem_g.at[b])

        def drain_gather(b):
            pltpu.make_async_copy(v_hbm.at[pl.ds(0, _CH)], vbufs.at[b],
                                  sem_g.at[b]).wait()

        def scatter(chunk, b):
            base = wid * _RPW + chunk * _CH
            pltpu.async_copy(vbufs.at[b], vout.at[pl.ds(base, _CH)],
                             sem_s.at[b])

        def drain_scatter(b):
            pltpu.make_async_copy(vbufs.at[b], vout.at[pl.ds(0, _CH)],
                                  sem_s.at[b]).wait()

        for b in range(_NBUF):
            fire(b, b)

        def group(g, carry):
            for b in range(_NBUF):
                drain_gather(b)
                scatter(g * _NBUF + b, b)

            @pl.when(g < _NGRP - 1)
            def _():
                for b in range(_NBUF):
                    drain_scatter(b)
                    fire((g + 1) * _NBUF + b, b)

            return carry

        lax.fori_loop(0, _NGRP, group, 0)
        for b in range(_NBUF):
            drain_scatter(b)

    return gather_kernel(v, idx)


def _attn_post_body(vn_ref, ccp_ref, feat_ref, wp1p_ref, bp1_ref,
                    wp2_ref, bp2_ref, ww2_ref, e8_ref, g1_ref, b1_ref, wl_ref,
                    g2_ref, b2_ref, out_ref,
                    att_vm, y_vm, as_vm, ass_vm, ys_vm, yss_vm):
    p = pl.program_id(0)
    i = pl.program_id(1)
    rows = pl.ds(i * _B, _B)
    half = _D // 2

    @pl.when(p == 0)
    def _attention():
        row = vn_ref[...]
        ccp = ccp_ref[...]
        ccp_rep = jnp.broadcast_to(
            ccp[:, None, :], (_B, _K, _D)).reshape(_BK, _D)
        h = jnp.maximum(
            jnp.dot(row, wp1p_ref[...], preferred_element_type=jnp.float32)
            - ccp_rep + bp1_ref[...], 0.0)
        logits = (jnp.dot(h, ww2_ref[...],
                          preferred_element_type=jnp.float32)
                  - row[:, half:half + _G]).reshape(_B, _K, _G)
        m = jnp.max(logits, axis=1, keepdims=True)
        e = jnp.exp(logits - m)
        denom = jnp.sum(e, axis=1)
        pe = jnp.dot(h, wp2_ref[...],
                     preferred_element_type=jnp.float32) + bp2_ref[...]
        wf = jnp.dot(e.reshape(_BK, _G), e8_ref[...],
                     preferred_element_type=jnp.float32)
        pk = lax.bitcast_convert_type(row[:, :half], jnp.int32)
        vhi = lax.bitcast_convert_type(pk & jnp.int32(-65536), jnp.float32)
        vlo = lax.bitcast_convert_type(pk << 16, jnp.float32)
        auL = jnp.sum((wf[:, :half] * (vhi + pe[:, :half])
                       ).reshape(_B, _K, half), axis=1)
        auH = jnp.sum((wf[:, half:] * (vlo + pe[:, half:])
                       ).reshape(_B, _K, half), axis=1)
        att = jnp.concatenate([auL, auH], axis=-1) / jnp.dot(
            denom, e8_ref[...], preferred_element_type=jnp.float32)
        att_vm[rows, :] = att

        @pl.when(i == 0)
        def _():
            as_vm[...] = jnp.zeros_like(as_vm)
            ass_vm[...] = jnp.zeros_like(ass_vm)

        as_vm[...] += jnp.sum(att, axis=0, keepdims=True)
        ass_vm[...] += jnp.sum(att * att, axis=0, keepdims=True)

    @pl.when(p == 1)
    def _post():
        mean, inv = _bn_scale(as_vm[...], ass_vm[...], g1_ref[...])
        f1 = jnp.maximum((att_vm[rows, :] - mean) * inv + b1_ref[...], 0.0)
        y = jnp.dot(f1, wl_ref[...], preferred_element_type=jnp.float32)
        y_vm[rows, :] = y

        @pl.when(i == 0)
        def _():
            ys_vm[...] = jnp.zeros_like(ys_vm)
            yss_vm[...] = jnp.zeros_like(yss_vm)

        ys_vm[...] += jnp.sum(y, axis=0, keepdims=True)
        yss_vm[...] += jnp.sum(y * y, axis=0, keepdims=True)

    @pl.when(p == 2)
    def _final():
        mean, inv = _bn_scale(ys_vm[...], yss_vm[...], g2_ref[...])
        out_ref[...] = jnp.maximum(
            feat_ref[...] + (y_vm[rows, :] - mean) * inv + b2_ref[...], 0.0)


def _phase_spec(bs, phase):
    return pl.BlockSpec(bs, lambda p, i: (jnp.where(p == phase, i, 0), 0))


def _out_spec(bs, phase):
    return pl.BlockSpec(bs, lambda p, i: (jnp.where(p == phase, i, 0), 0))


def _const_spec(bs):
    return pl.BlockSpec(bs, lambda p, i: (0, 0))


_ARB2 = pltpu.CompilerParams(
    dimension_semantics=("arbitrary", "arbitrary"))


def kernel(coords, features, neighbor_indices, W_proj, g_proj, b_proj,
           Wq, bq, Wk, bk, Wv, bv, Wp1, bp1, Wp2, bp2, Ww, bw,
           g1, b1, W_lin, g2, b2):
    f32 = jnp.float32
    wkw = Wk @ Ww
    wkw_pad = jnp.pad(wkw, ((0, 0), (0, 16 - _G)))
    bkw_pad = jnp.pad(bk @ Ww, (0, 16 - _G)).reshape(1, 16)
    cpad = jnp.pad(coords.astype(f32), ((0, 0), (_G, 16 - _G - 3)))
    wp1_pad = jnp.zeros((16, _D), f32).at[_G:_G + 3, :].set(Wp1)
    wp1_pad128 = jnp.zeros((_D, _D), f32).at[_D // 2 + _G:
                                             _D // 2 + _G + 3, :].set(Wp1)
    ww2 = Wp2 @ Ww
    e8 = (jnp.arange(_D)[None, :] // (_D // _G)
          == jnp.arange(_G)[:, None]).astype(f32)
    r = lambda a: a.reshape(1, -1)

    v, ccp = pl.pallas_call(
        _proj_qkv_body,
        grid=(2, _NB),
        in_specs=[_phase_spec((_B, _D), 0), _const_spec((_D, _D)),
                  _phase_spec((_B, 16), 1), _const_spec((1, _D)),
                  _const_spec((1, _D)), _const_spec((_D, _D)),
                  _const_spec((1, _D)), _const_spec((_D, 16)),
                  _const_spec((1, 16)), _const_spec((16, _D))],
        out_specs=[_out_spec((_B, _D), 1), _out_spec((_B, _D), 1)],
        out_shape=[jax.ShapeDtypeStruct((_N, _D), f32),
                   jax.ShapeDtypeStruct((_N, _D), f32)],
        scratch_shapes=[pltpu.VMEM((_N, _D), f32),
                        pltpu.VMEM((1, _D), f32),
                        pltpu.VMEM((1, _D), f32)],
        compiler_params=_ARB2,
    )(features, W_proj, cpad, r(g_proj), r(b_proj), Wv, r(bv), wkw_pad,
      bkw_pad, wp1_pad)

    idx = neighbor_indices.astype(jnp.int32).reshape(-1)
    vn = _sc_gather(v, idx)

    out = pl.pallas_call(
        _attn_post_body,
        grid=(3, _NB),
        in_specs=[_phase_spec((_BK, _D), 0),
                  _phase_spec((_B, _D), 0), _phase_spec((_B, _D), 2),
                  _const_spec((_D, _D)), _const_spec((1, _D)),
                  _const_spec((_D, _D)), _const_spec((1, _D)),
                  _const_spec((_D, _G)), _const_spec((_G, _D)),
                  _const_spec((1, _D)), _const_spec((1, _D)),
                  _const_spec((_D, _D)), _const_spec((1, _D)),
                  _const_spec((1, _D))],
        out_specs=_out_spec((_B, _D), 2),
        out_shape=jax.ShapeDtypeStruct((_N, _D), f32),
        scratch_shapes=[pltpu.VMEM((_N, _D), f32),
                        pltpu.VMEM((_N, _D), f32),
                        pltpu.VMEM((1, _D), f32),
                        pltpu.VMEM((1, _D), f32),
                        pltpu.VMEM((1, _D), f32),
                        pltpu.VMEM((1, _D), f32)],
        compiler_params=_ARB2,
    )(vn, ccp, features, wp1_pad128, r(bp1), Wp2, r(bp2), ww2, e8,
      r(g1), r(b1), W_lin, r(g2), r(b2))
    return out
